# baseline (speedup 1.0000x reference)
import jax
import jax.numpy as jnp
from jax import lax
from jax.experimental import pallas as pl
from jax.experimental.pallas import tpu as pltpu
from jax.experimental.pallas import tpu_sc as plsc

NUM_CORES = 2
NUM_SUBCORES = 16
NW = NUM_CORES * NUM_SUBCORES
DIM = 64
BATCH = 16384
BPW = BATCH // NW
CHUNK = 128
NCHUNK = BPW // CHUNK

SCALE = jnp.float32(1.0 + 2.0 ** -30)


def _body(h_hbm, r_hbm, t_hbm, ent_hbm, rel_hbm, out_hbm,
          hidx_v, ridx_v, tidx_v, hrow_v, rrow_v, trow_v, out_v, sem):
    cid = lax.axis_index("c")
    sid = lax.axis_index("s")
    wid = sid * NUM_CORES + cid

    pltpu.sync_copy(h_hbm.at[wid], hidx_v)
    pltpu.sync_copy(r_hbm.at[wid], ridx_v)
    pltpu.sync_copy(t_hbm.at[wid], tidx_v)

    copies = []
    for j in range(NCHUNK):
        sl = pl.ds(j * CHUNK, CHUNK)
        copies.append(pltpu.async_copy(ent_hbm.at[hidx_v.at[j]], hrow_v.at[sl], sem))
        copies.append(pltpu.async_copy(rel_hbm.at[ridx_v.at[j]], rrow_v.at[sl], sem))
        copies.append(pltpu.async_copy(ent_hbm.at[tidx_v.at[j]], trow_v.at[sl], sem))
    for c in copies:
        c.wait()

    lane = lax.iota(jnp.int32, 16)

    def block(b, _):
        acc = jnp.zeros((16,), jnp.float32)
        for j in range(16):
            row = b * 16 + j
            s = jnp.zeros((16,), jnp.float32)
            for k in range(DIM // 16):
                sl = pl.ds(k * 16, 16)
                s = s + jnp.abs(hrow_v[row, sl] + rrow_v[row, sl]
                                - trow_v[row, sl])
            tot = jnp.sum(s)
            acc = jnp.where(lane == j, -tot, acc)
        out_v[pl.ds(b * 16, 16)] = acc
        return 0

    lax.fori_loop(0, BPW // 16, block, 0)

    pltpu.sync_copy(out_v, out_hbm.at[pl.ds(wid * BPW, BPW)])


@jax.jit
def kernel(h, r, t, ent_weight, rel_weight):
    h3 = h.astype(jnp.int32).reshape(NW, NCHUNK, CHUNK)
    r3 = r.astype(jnp.int32).reshape(NW, NCHUNK, CHUNK)
    t3 = t.astype(jnp.int32).reshape(NW, NCHUNK, CHUNK)
    ent2 = ent_weight * SCALE
    rel2 = rel_weight * SCALE

    run = pl.kernel(
        _body,
        out_type=jax.ShapeDtypeStruct((BATCH,), jnp.float32),
        mesh=plsc.VectorSubcoreMesh(core_axis_name="c", subcore_axis_name="s"),
        compiler_params=pltpu.CompilerParams(
            needs_layout_passes=False, use_tc_tiling_on_sc=False),
        scratch_types=[
            pltpu.VMEM((NCHUNK, CHUNK), jnp.int32),
            pltpu.VMEM((NCHUNK, CHUNK), jnp.int32),
            pltpu.VMEM((NCHUNK, CHUNK), jnp.int32),
            pltpu.VMEM((BPW, DIM), jnp.float32),
            pltpu.VMEM((BPW, DIM), jnp.float32),
            pltpu.VMEM((BPW, DIM), jnp.float32),
            pltpu.VMEM((BPW,), jnp.float32),
            pltpu.SemaphoreType.DMA,
        ],
    )
    return run(h3, r3, t3, ent2, rel2)


# speedup vs baseline: 1.6470x; 1.6470x over previous
import jax
import jax.numpy as jnp
from jax import lax
from jax.experimental import pallas as pl
from jax.experimental.pallas import tpu as pltpu
from jax.experimental.pallas import tpu_sc as plsc

NUM_CORES = 2
NUM_SUBCORES = 16
NW = NUM_CORES * NUM_SUBCORES
DIM = 64
BATCH = 16384
BPW = BATCH // NW
CH = 32
NCH = BPW // CH

SCALE = jnp.float32(1.0 + 2.0 ** -30)


def _body(h_hbm, r_hbm, t_hbm, ent_hbm, rel_hbm, out_hbm,
          hidx_v, ridx_v, tidx_v, hrow_v, rrow_v, trow_v, out_v, sem):
    cid = lax.axis_index("c")
    sid = lax.axis_index("s")
    wid = sid * NUM_CORES + cid
    base = wid * BPW

    pltpu.sync_copy(h_hbm.at[pl.ds(base, BPW)], hidx_v)
    pltpu.sync_copy(r_hbm.at[pl.ds(base, BPW)], ridx_v)
    pltpu.sync_copy(t_hbm.at[pl.ds(base, BPW)], tidx_v)

    lane = lax.iota(jnp.int32, 16)
    zero16 = jnp.zeros((16,), jnp.int32)

    def chunk(g, _):
        row0 = g * CH
        copies = []
        for v in range(CH // 16):
            hv = hidx_v[pl.ds(row0 + v * 16, 16)]
            rv = ridx_v[pl.ds(row0 + v * 16, 16)]
            tv = tidx_v[pl.ds(row0 + v * 16, 16)]
            for j in range(16):
                i = v * 16 + j
                copies.append(pltpu.async_copy(
                    ent_hbm.at[jnp.sum(jnp.where(lane == j, hv, zero16))],
                    hrow_v.at[i], sem))
                copies.append(pltpu.async_copy(
                    rel_hbm.at[jnp.sum(jnp.where(lane == j, rv, zero16))],
                    rrow_v.at[i], sem))
                copies.append(pltpu.async_copy(
                    ent_hbm.at[jnp.sum(jnp.where(lane == j, tv, zero16))],
                    trow_v.at[i], sem))
        for c in copies:
            c.wait()

        for b in range(CH // 16):
            acc = jnp.zeros((16,), jnp.float32)
            for j in range(16):
                rj = b * 16 + j
                s = jnp.zeros((16,), jnp.float32)
                for k in range(DIM // 16):
                    sl = pl.ds(k * 16, 16)
                    s = s + jnp.abs(hrow_v[rj, sl] + rrow_v[rj, sl]
                                    - trow_v[rj, sl])
                tot = jnp.sum(s)
                acc = jnp.where(lane == j, -tot, acc)
            out_v[pl.ds(row0 + b * 16, 16)] = acc
        return 0

    lax.fori_loop(0, NCH, chunk, 0)

    pltpu.sync_copy(out_v, out_hbm.at[pl.ds(base, BPW)])


@jax.jit
def kernel(h, r, t, ent_weight, rel_weight):
    h1 = h.astype(jnp.int32)
    r1 = r.astype(jnp.int32)
    t1 = t.astype(jnp.int32)
    ent2 = ent_weight * SCALE
    rel2 = rel_weight * SCALE

    run = pl.kernel(
        _body,
        out_type=jax.ShapeDtypeStruct((BATCH,), jnp.float32),
        mesh=plsc.VectorSubcoreMesh(core_axis_name="c", subcore_axis_name="s"),
        compiler_params=pltpu.CompilerParams(needs_layout_passes=False),
        scratch_types=[
            pltpu.VMEM((BPW,), jnp.int32),
            pltpu.VMEM((BPW,), jnp.int32),
            pltpu.VMEM((BPW,), jnp.int32),
            pltpu.VMEM((CH, DIM), jnp.float32),
            pltpu.VMEM((CH, DIM), jnp.float32),
            pltpu.VMEM((CH, DIM), jnp.float32),
            pltpu.VMEM((BPW,), jnp.float32),
            pltpu.SemaphoreType.DMA,
        ],
    )
    return run(h1, r1, t1, ent2, rel2)


# speedup vs baseline: 1.6526x; 1.0034x over previous
import functools

import jax
import jax.numpy as jnp
from jax import lax
from jax.experimental import pallas as pl
from jax.experimental.pallas import tpu as pltpu
from jax.experimental.pallas import tpu_sc as plsc
from jax.experimental.layout import Format, Layout

NUM_CORES = 2
NUM_SUBCORES = 16
NW = NUM_CORES * NUM_SUBCORES
DIM = 64
BATCH = 16384
BPW = BATCH // NW
CH = 32
NCH = BPW // CH

SCALE = jnp.float32(1.0 + 2.0 ** -30)


def _body(h_hbm, r_hbm, t_hbm, ent_hbm, rel_hbm, out_hbm,
          hidx_v, ridx_v, tidx_v, hrow_v, rrow_v, trow_v, out_v, sem):
    cid = lax.axis_index("c")
    sid = lax.axis_index("s")
    wid = sid * NUM_CORES + cid
    base = wid * BPW

    pltpu.sync_copy(h_hbm.at[pl.ds(base, BPW)], hidx_v)
    pltpu.sync_copy(r_hbm.at[pl.ds(base, BPW)], ridx_v)
    pltpu.sync_copy(t_hbm.at[pl.ds(base, BPW)], tidx_v)

    lane = lax.iota(jnp.int32, 16)
    zero16 = jnp.zeros((16,), jnp.int32)

    def chunk(g, _):
        row0 = g * CH
        copies = []
        for v in range(CH // 16):
            hv = hidx_v[pl.ds(row0 + v * 16, 16)]
            rv = ridx_v[pl.ds(row0 + v * 16, 16)]
            tv = tidx_v[pl.ds(row0 + v * 16, 16)]
            for j in range(16):
                i = v * 16 + j
                copies.append(pltpu.async_copy(
                    ent_hbm.at[jnp.sum(jnp.where(lane == j, hv, zero16))],
                    hrow_v.at[i], sem))
                copies.append(pltpu.async_copy(
                    rel_hbm.at[jnp.sum(jnp.where(lane == j, rv, zero16))],
                    rrow_v.at[i], sem))
                copies.append(pltpu.async_copy(
                    ent_hbm.at[jnp.sum(jnp.where(lane == j, tv, zero16))],
                    trow_v.at[i], sem))
        for c in copies:
            c.wait()

        for b in range(CH // 16):
            acc = jnp.zeros((16,), jnp.float32)
            for j in range(16):
                rj = b * 16 + j
                s = jnp.zeros((16,), jnp.float32)
                for k in range(DIM // 16):
                    sl = pl.ds(k * 16, 16)
                    s = s + jnp.abs(hrow_v[rj, sl] + rrow_v[rj, sl]
                                    - trow_v[rj, sl])
                tot = jnp.sum(s)
                acc = jnp.where(lane == j, -tot, acc)
            out_v[pl.ds(row0 + b * 16, 16)] = acc
        return 0

    lax.fori_loop(0, NCH, chunk, 0)

    pltpu.sync_copy(out_v, out_hbm.at[pl.ds(base, BPW)])


@functools.lru_cache(maxsize=1)
def _jitted():
    import jax.sharding as jsh
    dev = jax.devices()[0]
    single = jsh.SingleDeviceSharding(dev)
    dense2d = Format(Layout(major_to_minor=(1, 0), tiling=()), single)
    dflt = Format(None, single)
    return jax.jit(
        _kernel_impl,
        in_shardings=(dflt, dflt, dflt, dense2d, dense2d),
    )


def kernel(h, r, t, ent_weight, rel_weight):
    return _jitted()(h, r, t, ent_weight, rel_weight)


def _kernel_impl(h, r, t, ent_weight, rel_weight):
    h1 = h.astype(jnp.int32)
    r1 = r.astype(jnp.int32)
    t1 = t.astype(jnp.int32)
    ent2 = ent_weight
    rel2 = rel_weight

    run = pl.kernel(
        _body,
        out_type=jax.ShapeDtypeStruct((BATCH,), jnp.float32),
        mesh=plsc.VectorSubcoreMesh(core_axis_name="c", subcore_axis_name="s"),
        compiler_params=pltpu.CompilerParams(needs_layout_passes=False),
        scratch_types=[
            pltpu.VMEM((BPW,), jnp.int32),
            pltpu.VMEM((BPW,), jnp.int32),
            pltpu.VMEM((BPW,), jnp.int32),
            pltpu.VMEM((CH, DIM), jnp.float32),
            pltpu.VMEM((CH, DIM), jnp.float32),
            pltpu.VMEM((CH, DIM), jnp.float32),
            pltpu.VMEM((BPW,), jnp.float32),
            pltpu.SemaphoreType.DMA,
        ],
    )
    return run(h1, r1, t1, ent2, rel2)
